# baseline (speedup 1.0000x reference)
import functools

import jax
import jax.numpy as jnp
from jax import lax
from jax.experimental import pallas as pl
from jax.experimental.pallas import tpu as pltpu
from jax.experimental.pallas import tpu_sc as plsc

_ETA = 0.5
_L = 16
_ROW = 32
_NW = 32
_TB = 128


def _swap_pairs(x):
    perm = lax.iota(jnp.int32, _L) ^ 1
    dnums = lax.GatherDimensionNumbers(
        offset_dims=(), collapsed_slice_dims=(0,), start_index_map=(0,))
    return lax.gather(x, perm[:, None], dnums, (1,),
                      mode=lax.GatherScatterMode.PROMISE_IN_BOUNDS)


def _rsqrt(s):
    i = lax.bitcast_convert_type(s, jnp.int32)
    y = lax.bitcast_convert_type(0x5F3759DF - (i >> 1), jnp.float32)
    for _ in range(3):
        y = y * (1.5 - 0.5 * s * y * y)
    return y


def _sc_body(field_hbm, idx_hbm, tlo_hbm, thi_hbm, out_hbm,
             idx_v, fld_v, out_v, blo_v, bhi_v, sem):
    nc = 2
    wid = lax.axis_index("s") * nc + lax.axis_index("c")
    n_heads = 8
    tpw = 4

    pltpu.sync_copy(idx_hbm.at[pl.ds(wid * (tpw * n_heads), tpw * n_heads)],
                    idx_v)
    pltpu.sync_copy(field_hbm.at[:, pl.ds(wid * tpw, tpw)], fld_v)

    iota = lax.iota(jnp.int32, _L)
    d_vec = iota >> 1
    c_vec = iota & 1

    for tloc in range(tpw):
        copies = [
            pltpu.async_copy(
                t_hbm.at[idx_v.at[tloc * n_heads + h]],
                b_v.at[pl.ds(h * _TB, _TB), :], sem)
            for h in range(n_heads)
            for t_hbm, b_v in ((tlo_hbm, blo_v), (thi_hbm, bhi_v))
        ]
        for cp in copies:
            cp.wait()

        t_vec = jnp.full((_L,), tloc, jnp.int32)

        def row(l, carry, tloc=tloc):
            l_vec = jnp.full((_L,), 0, jnp.int32) + l
            r0 = jnp.full((_L,), 0, jnp.int32) + l
            a0 = plsc.load_gather(blo_v, [r0, iota])
            a1 = plsc.load_gather(bhi_v, [r0, iota])
            for h in range(1, n_heads):
                rh = r0 + (h * _TB)
                a0 = a0 + plsc.load_gather(blo_v, [rh, iota])
                a1 = a1 + plsc.load_gather(bhi_v, [rh, iota])
            f0 = plsc.load_gather(fld_v, [d_vec, t_vec, c_vec, l_vec])
            f1 = plsc.load_gather(fld_v, [d_vec + 8, t_vec, c_vec, l_vec])
            n0 = (1.0 - _ETA) * f0 + (_ETA / n_heads) * a0
            n1 = (1.0 - _ETA) * f1 + (_ETA / n_heads) * a1
            for nv, dv in ((n0, d_vec), (n1, d_vec + 8)):
                sq = nv * nv
                s = sq + _swap_pairs(sq)
                inv = jnp.minimum(_rsqrt(s), 1e6)
                plsc.store_scatter(out_v, [dv, t_vec, c_vec, l_vec], nv * inv)
            return carry
        lax.fori_loop(0, _TB, row, 0)

    pltpu.sync_copy(out_v, out_hbm.at[:, pl.ds(wid * tpw, tpw)])


@functools.partial(jax.jit, static_argnums=())
def _sc_call(field_p, idx2, table_lo, table_hi):
    mesh = plsc.VectorSubcoreMesh(core_axis_name="c", subcore_axis_name="s")
    tpw = field_p.shape[1] // _NW
    fn = pl.kernel(
        _sc_body,
        out_type=jax.ShapeDtypeStruct(field_p.shape, jnp.float32),
        mesh=mesh,
        scratch_types=[
            pltpu.VMEM((idx2.shape[0] // _NW, _TB), jnp.int32),
            pltpu.VMEM((16, tpw, 2, _TB), jnp.float32),
            pltpu.VMEM((16, tpw, 2, _TB), jnp.float32),
            pltpu.VMEM((8 * _TB, _L), jnp.float32),
            pltpu.VMEM((8 * _TB, _L), jnp.float32),
            pltpu.SemaphoreType.DMA,
        ],
        compiler_params=pltpu.CompilerParams(
            use_tc_tiling_on_sc=False, needs_layout_passes=False),
    )
    return fn(field_p, idx2, table_lo, table_hi)


def kernel(field, idx_r, bucket_code):
    b, d, _ = field.shape
    buckets = bucket_code.shape[0]
    nb = b // _TB
    field_p = field.reshape(nb, _TB, d, 2).transpose(2, 0, 3, 1)
    idx = jnp.clip(idx_r[:, -1, :], 0, buckets - 1).astype(jnp.int32)
    idx2 = idx.reshape(nb, _TB, -1).transpose(0, 2, 1).reshape(-1, _TB)
    table_lo = bucket_code[:, :d // 2, :].reshape(buckets, d)
    table_hi = bucket_code[:, d // 2:, :].reshape(buckets, d)
    out_p = _sc_call(field_p, idx2, table_lo, table_hi)
    return out_p.transpose(1, 3, 0, 2).reshape(b, d, 2)


# speedup vs baseline: 1.4021x; 1.4021x over previous
import functools

import jax
import jax.numpy as jnp
from jax import lax
from jax.experimental import pallas as pl
from jax.experimental.pallas import tpu as pltpu
from jax.experimental.pallas import tpu_sc as plsc

_ETA = 0.5
_L = 16
_ROW = 32
_NW = 32
_TB = 128


def _swap_pairs(x):
    perm = lax.iota(jnp.int32, _L) ^ 1
    dnums = lax.GatherDimensionNumbers(
        offset_dims=(), collapsed_slice_dims=(0,), start_index_map=(0,))
    return lax.gather(x, perm[:, None], dnums, (1,),
                      mode=lax.GatherScatterMode.PROMISE_IN_BOUNDS)


def _rsqrt(s):
    i = lax.bitcast_convert_type(s, jnp.int32)
    y = lax.bitcast_convert_type(0x5F3759DF - (i >> 1), jnp.float32)
    for _ in range(3):
        y = y * (1.5 - 0.5 * s * y * y)
    return y


def _sc_body(field_hbm, idx_hbm, table_hbm, out_hbm,
             idx_v, fld_v, out_v, buf_v, sem):
    nc = 2
    wid = lax.axis_index("s") * nc + lax.axis_index("c")
    n_heads = 8
    tpw = 4

    pltpu.sync_copy(idx_hbm.at[pl.ds(wid * (tpw * n_heads), tpw * n_heads)],
                    idx_v)
    pltpu.sync_copy(field_hbm.at[:, pl.ds(wid * tpw, tpw)], fld_v)

    iota = lax.iota(jnp.int32, _L)
    d_vec = iota >> 1
    c_vec = iota & 1
    col_lo = iota
    col_hi = iota + _L

    for tloc in range(tpw):
        copies = [
            pltpu.async_copy(
                table_hbm.at[idx_v.at[tloc * n_heads + h]],
                buf_v.at[pl.ds(h * _TB, _TB), :], sem)
            for h in range(n_heads)
        ]
        for cp in copies:
            cp.wait()

        t_vec = jnp.full((_L,), tloc, jnp.int32)

        def row(l, carry, tloc=tloc):
            l_vec = jnp.full((_L,), 0, jnp.int32) + l
            r0 = jnp.full((_L,), 0, jnp.int32) + l
            a0 = plsc.load_gather(buf_v, [r0, col_lo])
            a1 = plsc.load_gather(buf_v, [r0, col_hi])
            for h in range(1, n_heads):
                rh = r0 + (h * _TB)
                a0 = a0 + plsc.load_gather(buf_v, [rh, col_lo])
                a1 = a1 + plsc.load_gather(buf_v, [rh, col_hi])
            f0 = plsc.load_gather(fld_v, [d_vec, t_vec, c_vec, l_vec])
            f1 = plsc.load_gather(fld_v, [d_vec + 8, t_vec, c_vec, l_vec])
            n0 = (1.0 - _ETA) * f0 + (_ETA / n_heads) * a0
            n1 = (1.0 - _ETA) * f1 + (_ETA / n_heads) * a1
            for nv, dv in ((n0, d_vec), (n1, d_vec + 8)):
                sq = nv * nv
                s = sq + _swap_pairs(sq)
                inv = jnp.minimum(_rsqrt(s), 1e6)
                plsc.store_scatter(out_v, [dv, t_vec, c_vec, l_vec], nv * inv)
            return carry
        lax.fori_loop(0, _TB, row, 0)

    pltpu.sync_copy(out_v, out_hbm.at[:, pl.ds(wid * tpw, tpw)])


@functools.partial(jax.jit, static_argnums=())
def _sc_call(field_p, idx2, table):
    mesh = plsc.VectorSubcoreMesh(core_axis_name="c", subcore_axis_name="s")
    tpw = field_p.shape[1] // _NW
    fn = pl.kernel(
        _sc_body,
        out_type=jax.ShapeDtypeStruct(field_p.shape, jnp.float32),
        mesh=mesh,
        scratch_types=[
            pltpu.VMEM((idx2.shape[0] // _NW, _TB), jnp.int32),
            pltpu.VMEM((16, tpw, 2, _TB), jnp.float32),
            pltpu.VMEM((16, tpw, 2, _TB), jnp.float32),
            pltpu.VMEM((8 * _TB, _ROW), jnp.float32),
            pltpu.SemaphoreType.DMA,
        ],
        compiler_params=pltpu.CompilerParams(
            use_tc_tiling_on_sc=False, needs_layout_passes=False),
    )
    return fn(field_p, idx2, table)


def kernel(field, idx_r, bucket_code):
    b, d, _ = field.shape
    buckets = bucket_code.shape[0]
    nb = b // _TB
    field_p = field.reshape(nb, _TB, d, 2).transpose(2, 0, 3, 1)
    idx = jnp.clip(idx_r[:, -1, :], 0, buckets - 1).astype(jnp.int32)
    idx2 = idx.reshape(nb, _TB, -1).transpose(0, 2, 1).reshape(-1, _TB)
    table = bucket_code.reshape(buckets, d * 2)
    out_p = _sc_call(field_p, idx2, table)
    return out_p.transpose(1, 3, 0, 2).reshape(b, d, 2)


# speedup vs baseline: 1.4150x; 1.0093x over previous
import functools

import jax
import jax.numpy as jnp
from jax import lax
from jax.experimental import pallas as pl
from jax.experimental.pallas import tpu as pltpu
from jax.experimental.pallas import tpu_sc as plsc

_ETA = 0.5
_L = 16
_ROW = 32
_NW = 32
_TB = 128


def _swap_pairs(x):
    perm = lax.iota(jnp.int32, _L) ^ 1
    dnums = lax.GatherDimensionNumbers(
        offset_dims=(), collapsed_slice_dims=(0,), start_index_map=(0,))
    return lax.gather(x, perm[:, None], dnums, (1,),
                      mode=lax.GatherScatterMode.PROMISE_IN_BOUNDS)


def _rsqrt(s):
    i = lax.bitcast_convert_type(s, jnp.int32)
    y = lax.bitcast_convert_type(0x5F3759DF - (i >> 1), jnp.float32)
    for _ in range(3):
        y = y * (1.5 - 0.5 * s * y * y)
    return y


def _sc_body(field_hbm, idx_hbm, table_hbm, out_hbm,
             idx_v, fld_v, out_v, buf_a, buf_b, sem_a, sem_b):
    nc = 2
    wid = lax.axis_index("s") * nc + lax.axis_index("c")
    n_heads = 8
    tpw = 4

    pltpu.sync_copy(idx_hbm.at[pl.ds(wid * (tpw * n_heads), tpw * n_heads)],
                    idx_v)
    pltpu.sync_copy(field_hbm.at[:, pl.ds(wid * tpw, tpw)], fld_v)

    iota = lax.iota(jnp.int32, _L)
    d_vec = iota >> 1
    c_vec = iota & 1
    col_lo = iota
    col_hi = iota + _L

    bufs = (buf_a, buf_b)
    sems = (sem_a, sem_b)

    def fire(tloc):
        return [
            pltpu.async_copy(
                table_hbm.at[idx_v.at[tloc * n_heads + h]],
                bufs[tloc % 2].at[pl.ds(h * _TB, _TB), :], sems[tloc % 2])
            for h in range(n_heads)
        ]

    pending = fire(0)
    for tloc in range(tpw):
        for cp in pending:
            cp.wait()
        if tloc + 1 < tpw:
            pending = fire(tloc + 1)
        buf_v = bufs[tloc % 2]

        t_vec = jnp.full((_L,), tloc, jnp.int32)

        def row(l, carry, tloc=tloc, buf_v=buf_v):
            l_vec = jnp.full((_L,), 0, jnp.int32) + l
            r0 = jnp.full((_L,), 0, jnp.int32) + l
            a0 = plsc.load_gather(buf_v, [r0, col_lo])
            a1 = plsc.load_gather(buf_v, [r0, col_hi])
            for h in range(1, n_heads):
                rh = r0 + (h * _TB)
                a0 = a0 + plsc.load_gather(buf_v, [rh, col_lo])
                a1 = a1 + plsc.load_gather(buf_v, [rh, col_hi])
            f0 = plsc.load_gather(fld_v, [d_vec, t_vec, c_vec, l_vec])
            f1 = plsc.load_gather(fld_v, [d_vec + 8, t_vec, c_vec, l_vec])
            n0 = (1.0 - _ETA) * f0 + (_ETA / n_heads) * a0
            n1 = (1.0 - _ETA) * f1 + (_ETA / n_heads) * a1
            for nv, dv in ((n0, d_vec), (n1, d_vec + 8)):
                sq = nv * nv
                s = sq + _swap_pairs(sq)
                inv = jnp.minimum(_rsqrt(s), 1e6)
                plsc.store_scatter(out_v, [dv, t_vec, c_vec, l_vec], nv * inv)
            return carry
        lax.fori_loop(0, _TB, row, 0)

    pltpu.sync_copy(out_v, out_hbm.at[:, pl.ds(wid * tpw, tpw)])


@functools.partial(jax.jit, static_argnums=())
def _sc_call(field_p, idx2, table):
    mesh = plsc.VectorSubcoreMesh(core_axis_name="c", subcore_axis_name="s")
    tpw = field_p.shape[1] // _NW
    fn = pl.kernel(
        _sc_body,
        out_type=jax.ShapeDtypeStruct(field_p.shape, jnp.float32),
        mesh=mesh,
        scratch_types=[
            pltpu.VMEM((idx2.shape[0] // _NW, _TB), jnp.int32),
            pltpu.VMEM((16, tpw, 2, _TB), jnp.float32),
            pltpu.VMEM((16, tpw, 2, _TB), jnp.float32),
            pltpu.VMEM((8 * _TB, _ROW), jnp.float32),
            pltpu.VMEM((8 * _TB, _ROW), jnp.float32),
            pltpu.SemaphoreType.DMA,
            pltpu.SemaphoreType.DMA,
        ],
        compiler_params=pltpu.CompilerParams(
            use_tc_tiling_on_sc=False, needs_layout_passes=False),
    )
    return fn(field_p, idx2, table)


def kernel(field, idx_r, bucket_code):
    b, d, _ = field.shape
    buckets = bucket_code.shape[0]
    nb = b // _TB
    field_p = field.reshape(nb, _TB, d, 2).transpose(2, 0, 3, 1)
    idx = jnp.clip(idx_r[:, -1, :], 0, buckets - 1).astype(jnp.int32)
    idx2 = idx.reshape(nb, _TB, -1).transpose(0, 2, 1).reshape(-1, _TB)
    table = bucket_code.reshape(buckets, d * 2)
    out_p = _sc_call(field_p, idx2, table)
    return out_p.transpose(1, 3, 0, 2).reshape(b, d, 2)
